# Initial kernel scaffold; baseline (speedup 1.0000x reference)
import jax
import jax.numpy as jnp
from jax.experimental import pallas as pl


def kernel(x, edge_index, edge_type, W_rgcn, root, b_rgcn, Wq, bq, Wk, bk, Wv, bv, Wskip, bskip, Wres, bres, gamma, beta):
    raise NotImplementedError("write your pallas kernel here")


# speedup vs baseline: 1.4253x; 1.4253x over previous
import functools

import jax
import jax.numpy as jnp
from jax.experimental import pallas as pl
from jax.experimental.pallas import tpu as pltpu

_N = 10000
_E = 160000
_D = 256
_R = 6
_BLK = 1000
_NB = _N // _BLK


def _mm1_body(x_ref, W_ref, root_ref, b_ref, xw_ref, hpre_ref):
    x = x_ref[...]
    for r in range(_R):
        xw_ref[r] = jnp.dot(x, W_ref[r], preferred_element_type=jnp.float32)
    hpre_ref[...] = (
        jnp.dot(x, root_ref[...], preferred_element_type=jnp.float32)
        + b_ref[...]
    )


def _mm1(x, W_rgcn, root, b_rgcn):
    b2 = b_rgcn.reshape(1, _D)
    return pl.pallas_call(
        _mm1_body,
        grid=(_NB,),
        in_specs=[
            pl.BlockSpec((_BLK, _D), lambda i: (i, 0)),
            pl.BlockSpec((_R, _D, _D), lambda i: (0, 0, 0)),
            pl.BlockSpec((_D, _D), lambda i: (0, 0)),
            pl.BlockSpec((1, _D), lambda i: (0, 0)),
        ],
        out_specs=[
            pl.BlockSpec((_R, _BLK, _D), lambda i: (0, i, 0)),
            pl.BlockSpec((_BLK, _D), lambda i: (i, 0)),
        ],
        out_shape=[
            jax.ShapeDtypeStruct((_R, _N, _D), jnp.float32),
            jax.ShapeDtypeStruct((_N, _D), jnp.float32),
        ],
    )(x, W_rgcn, root, b2)


def _mm2_body(agg_ref, hpre_ref, Wq_ref, bq_ref, Wk_ref, bk_ref, Wv_ref,
              bv_ref, Ws_ref, bs_ref, Wr_ref, br_ref,
              q_ref, k_ref, v_ref, base_ref):
    h = jnp.maximum(agg_ref[...] + hpre_ref[...], 0.0)
    q_ref[...] = jnp.dot(h, Wq_ref[...], preferred_element_type=jnp.float32) + bq_ref[...]
    k_ref[...] = jnp.dot(h, Wk_ref[...], preferred_element_type=jnp.float32) + bk_ref[...]
    v_ref[...] = jnp.dot(h, Wv_ref[...], preferred_element_type=jnp.float32) + bv_ref[...]
    base_ref[...] = (
        jnp.dot(h, Ws_ref[...] + Wr_ref[...], preferred_element_type=jnp.float32)
        + bs_ref[...] + br_ref[...]
    )


def _mm2(agg, hpre, Wq, bq, Wk, bk, Wv, bv, Wskip, bskip, Wres, bres):
    wspec = pl.BlockSpec((_D, _D), lambda i: (0, 0))
    bspec = pl.BlockSpec((1, _D), lambda i: (0, 0))
    nspec = pl.BlockSpec((_BLK, _D), lambda i: (i, 0))
    return pl.pallas_call(
        _mm2_body,
        grid=(_NB,),
        in_specs=[nspec, nspec] + [wspec, bspec] * 5,
        out_specs=[nspec] * 4,
        out_shape=[jax.ShapeDtypeStruct((_N, _D), jnp.float32)] * 4,
    )(agg, hpre, Wq, bq.reshape(1, _D), Wk, bk.reshape(1, _D),
      Wv, bv.reshape(1, _D), Wskip, bskip.reshape(1, _D),
      Wres, bres.reshape(1, _D))


def _fin1_body(numer_ref, denom_ref, base_ref, o_ref, bsum_ref, bsq_ref):
    d = denom_ref[...]
    attn = numer_ref[...] / (d + 1e-16)
    o = jnp.maximum(attn + base_ref[...], 0.0)
    o_ref[...] = o
    bsum_ref[...] = jnp.sum(o, axis=0, keepdims=True)[None]
    bsq_ref[...] = jnp.sum(o * o, axis=0, keepdims=True)[None]


def _fin1(numer, denom, base):
    nspec = pl.BlockSpec((_BLK, _D), lambda i: (i, 0))
    return pl.pallas_call(
        _fin1_body,
        grid=(_NB,),
        in_specs=[nspec,
                  pl.BlockSpec((_BLK, 1), lambda i: (i, 0)),
                  nspec],
        out_specs=[nspec,
                   pl.BlockSpec((1, 1, _D), lambda i: (i, 0, 0)),
                   pl.BlockSpec((1, 1, _D), lambda i: (i, 0, 0))],
        out_shape=[jax.ShapeDtypeStruct((_N, _D), jnp.float32),
                   jax.ShapeDtypeStruct((_NB, 1, _D), jnp.float32),
                   jax.ShapeDtypeStruct((_NB, 1, _D), jnp.float32)],
    )(numer, denom, base)


def _fin2_body(o_ref, bsum_ref, bsq_ref, gamma_ref, beta_ref, out_ref):
    s = jnp.sum(bsum_ref[...], axis=(0, 1), keepdims=False)[None]
    sq = jnp.sum(bsq_ref[...], axis=(0, 1), keepdims=False)[None]
    mean = s / _N
    var = sq / _N - mean * mean
    inv = jax.lax.rsqrt(var + 1e-5)
    out_ref[...] = (o_ref[...] - mean) * (inv * gamma_ref[...]) + beta_ref[...]


def _fin2(o, bsum, bsq, gamma, beta):
    nspec = pl.BlockSpec((_BLK, _D), lambda i: (i, 0))
    sspec = pl.BlockSpec((_NB, 1, _D), lambda i: (0, 0, 0))
    bspec = pl.BlockSpec((1, _D), lambda i: (0, 0))
    return pl.pallas_call(
        _fin2_body,
        grid=(_NB,),
        in_specs=[nspec, sspec, sspec, bspec, bspec],
        out_specs=nspec,
        out_shape=jax.ShapeDtypeStruct((_N, _D), jnp.float32),
    )(o, bsum, bsq, gamma.reshape(1, _D), beta.reshape(1, _D))


def kernel(x, edge_index, edge_type, W_rgcn, root, b_rgcn, Wq, bq, Wk, bk,
           Wv, bv, Wskip, bskip, Wres, bres, gamma, beta):
    src = edge_index[0]
    tgt = edge_index[1]

    xw, hpre = _mm1(x, W_rgcn, root, b_rgcn)

    comb = tgt * _R + edge_type
    counts = jax.ops.segment_sum(jnp.ones((_E,), jnp.float32), comb,
                                 num_segments=_N * _R)
    norm = 1.0 / jnp.maximum(counts[comb], 1.0)
    msg = xw[edge_type, src]
    agg = jax.ops.segment_sum(msg * norm[:, None], tgt, num_segments=_N)

    q, k, v, base = _mm2(agg, hpre, Wq, bq, Wk, bk, Wv, bv,
                         Wskip, bskip, Wres, bres)

    score = jnp.sum(q[tgt] * k[src], axis=-1) * (1.0 / jnp.sqrt(jnp.float32(_D)))
    ex = jnp.exp(score)
    denom = jax.ops.segment_sum(ex, tgt, num_segments=_N)
    numer = jax.ops.segment_sum(ex[:, None] * v[src], tgt, num_segments=_N)

    o, bsum, bsq = _fin1(numer, denom.reshape(_N, 1), base)
    return _fin2(o, bsum, bsq, gamma, beta)


# speedup vs baseline: 2.0866x; 1.4640x over previous
import functools

import jax
import jax.numpy as jnp
from jax import lax
from jax.experimental import pallas as pl
from jax.experimental.pallas import tpu as pltpu
from jax.experimental.pallas import tpu_sc as plsc

_N = 10000
_NP = 10240
_E = 160000
_D = 256
_R = 6
_RN = _R * _N
_CF = 65536
_BLK = 1000
_NB = _N // _BLK

_NS = 16
_EC = _E // _NS
_B = 80
_NBLK = _EC // _B


def _mm1_body(x_ref, W_ref, root_ref, b_ref, xw_ref, hpre_ref):
    x = x_ref[...]
    for r in range(_R):
        xw = jnp.dot(x, W_ref[r], preferred_element_type=jnp.float32)
        xw_ref[0, r] = xw[:, :128]
        xw_ref[1, r] = xw[:, 128:]
    hpre_ref[...] = (
        jnp.dot(x, root_ref[...], preferred_element_type=jnp.float32)
        + b_ref[...]
    )


def _mm1(x, W_rgcn, root, b_rgcn):
    xw, hpre = pl.pallas_call(
        _mm1_body,
        grid=(_NB,),
        in_specs=[
            pl.BlockSpec((_BLK, _D), lambda i: (i, 0)),
            pl.BlockSpec((_R, _D, _D), lambda i: (0, 0, 0)),
            pl.BlockSpec((_D, _D), lambda i: (0, 0)),
            pl.BlockSpec((1, _D), lambda i: (0, 0)),
        ],
        out_specs=[
            pl.BlockSpec((2, _R, _BLK, 128), lambda i: (0, 0, i, 0)),
            pl.BlockSpec((_BLK, _D), lambda i: (i, 0)),
        ],
        out_shape=[
            jax.ShapeDtypeStruct((2, _R, _N, 128), jnp.float32),
            jax.ShapeDtypeStruct((_N, _D), jnp.float32),
        ],
    )(x, W_rgcn, root, b_rgcn.reshape(1, _D))
    return xw.reshape(2 * _RN, 128), hpre


def _mm2_body(a0_ref, a1_ref, hpre_ref, Wq_ref, bq_ref, Wk_ref, bk_ref,
              Wv_ref, bv_ref, Ws_ref, bs_ref, Wr_ref, br_ref,
              q_ref, k_ref, v_ref, base_ref):
    agg = jnp.concatenate([a0_ref[...], a1_ref[...]], axis=1)
    h = jnp.maximum(agg + hpre_ref[...], 0.0)
    q = jnp.dot(h, Wq_ref[...], preferred_element_type=jnp.float32) + bq_ref[...]
    k = jnp.dot(h, Wk_ref[...], preferred_element_type=jnp.float32) + bk_ref[...]
    v = jnp.dot(h, Wv_ref[...], preferred_element_type=jnp.float32) + bv_ref[...]
    q_ref[0], q_ref[1] = q[:, :128], q[:, 128:]
    k_ref[0], k_ref[1] = k[:, :128], k[:, 128:]
    v_ref[0], v_ref[1] = v[:, :128], v[:, 128:]
    base_ref[...] = (
        jnp.dot(h, Ws_ref[...] + Wr_ref[...], preferred_element_type=jnp.float32)
        + bs_ref[...] + br_ref[...]
    )


def _mm2(a0, a1, hpre, Wq, bq, Wk, bk, Wv, bv, Wskip, bskip, Wres, bres):
    wspec = pl.BlockSpec((_D, _D), lambda i: (0, 0))
    bspec = pl.BlockSpec((1, _D), lambda i: (0, 0))
    nspec = pl.BlockSpec((_BLK, _D), lambda i: (i, 0))
    hspec = pl.BlockSpec((_BLK, 128), lambda i: (i, 0))
    sspec = pl.BlockSpec((2, _BLK, 128), lambda i: (0, i, 0))
    q, k, v, base = pl.pallas_call(
        _mm2_body,
        grid=(_NB,),
        in_specs=[hspec, hspec, nspec] + [wspec, bspec] * 5,
        out_specs=[sspec, sspec, sspec, nspec],
        out_shape=[jax.ShapeDtypeStruct((2, _N, 128), jnp.float32)] * 3
        + [jax.ShapeDtypeStruct((_N, _D), jnp.float32)],
    )(a0, a1, hpre, Wq, bq.reshape(1, _D), Wk, bk.reshape(1, _D),
      Wv, bv.reshape(1, _D), Wskip, bskip.reshape(1, _D),
      Wres, bres.reshape(1, _D))
    return q, k, v, base


def _fin1_body(n0_ref, n1_ref, denom_ref, base_ref, o_ref, bsum_ref, bsq_ref):
    numer = jnp.concatenate([n0_ref[...], n1_ref[...]], axis=1)
    attn = numer / (denom_ref[...] + 1e-16)
    o = jnp.maximum(attn + base_ref[...], 0.0)
    o_ref[...] = o
    bsum_ref[...] = jnp.sum(o, axis=0, keepdims=True)[None]
    bsq_ref[...] = jnp.sum(o * o, axis=0, keepdims=True)[None]


def _fin1(n0, n1, denom, base):
    nspec = pl.BlockSpec((_BLK, _D), lambda i: (i, 0))
    hspec = pl.BlockSpec((_BLK, 128), lambda i: (i, 0))
    return pl.pallas_call(
        _fin1_body,
        grid=(_NB,),
        in_specs=[hspec, hspec,
                  pl.BlockSpec((_BLK, 1), lambda i: (i, 0)),
                  nspec],
        out_specs=[nspec,
                   pl.BlockSpec((1, 1, _D), lambda i: (i, 0, 0)),
                   pl.BlockSpec((1, 1, _D), lambda i: (i, 0, 0))],
        out_shape=[jax.ShapeDtypeStruct((_N, _D), jnp.float32),
                   jax.ShapeDtypeStruct((_NB, 1, _D), jnp.float32),
                   jax.ShapeDtypeStruct((_NB, 1, _D), jnp.float32)],
    )(n0, n1, denom, base)


def _fin2_body(o_ref, bsum_ref, bsq_ref, gamma_ref, beta_ref, out_ref):
    s = jnp.sum(bsum_ref[...], axis=(0, 1), keepdims=False)[None]
    sq = jnp.sum(bsq_ref[...], axis=(0, 1), keepdims=False)[None]
    mean = s / _N
    var = sq / _N - mean * mean
    inv = jax.lax.rsqrt(var + 1e-5)
    out_ref[...] = (o_ref[...] - mean) * (inv * gamma_ref[...]) + beta_ref[...]


def _fin2(o, bsum, bsq, gamma, beta):
    nspec = pl.BlockSpec((_BLK, _D), lambda i: (i, 0))
    sspec = pl.BlockSpec((_NB, 1, _D), lambda i: (0, 0, 0))
    bspec = pl.BlockSpec((1, _D), lambda i: (0, 0))
    return pl.pallas_call(
        _fin2_body,
        grid=(_NB,),
        in_specs=[nspec, sspec, sspec, bspec, bspec],
        out_specs=nspec,
        out_shape=jax.ShapeDtypeStruct((_N, _D), jnp.float32),
    )(o, bsum, bsq, gamma.reshape(1, _D), beta.reshape(1, _D))


def _sc1_body(src_hbm, tgt_hbm, et_hbm, xw_hbm, zrow_hbm, z1d_hbm, agg_hbm,
              srcb_v, tgtc_v, etc_v, combb_v, tgtb_v, gidx_v,
              cntb_v, normb_v, ones_v, rows_v, acc_sh, cnt_sh, sem):
    c = lax.axis_index("c")
    s = lax.axis_index("s")
    nslice = pl.ds(s * (_NP // _NS), _NP // _NS)
    cslice = pl.ds(s * (_CF // _NS), _CF // _NS)
    pltpu.sync_copy(zrow_hbm.at[nslice], acc_sh.at[nslice])
    pltpu.sync_copy(z1d_hbm.at[cslice], cnt_sh.at[cslice])
    ebase = s * _EC
    pltpu.sync_copy(tgt_hbm.at[pl.ds(ebase, _EC)], tgtc_v)
    pltpu.sync_copy(et_hbm.at[pl.ds(ebase, _EC)], etc_v)
    for g in range(_B // 16):
        ones_v[pl.ds(g * 16, 16)] = jnp.full((16,), 1.0, jnp.float32)
    plsc.subcore_barrier()

    def hist_step(b, carry):
        for g in range(_B // 16):
            sl = pl.ds(b * _B + g * 16, 16)
            comb16 = tgtc_v[sl] * _R + etc_v[sl]
            combb_v[pl.ds(g * 16, 16)] = comb16
        pltpu.sync_copy(ones_v, cnt_sh.at[combb_v], add=True)
        return carry

    lax.fori_loop(0, _NBLK, hist_step, 0)
    plsc.subcore_barrier()

    def blk_step(b, carry):
        pltpu.sync_copy(src_hbm.at[pl.ds(ebase + b * _B, _B)], srcb_v)
        for g in range(_B // 16):
            sl = pl.ds(b * _B + g * 16, 16)
            bsl = pl.ds(g * 16, 16)
            sv = srcb_v[bsl]
            tv = tgtc_v[sl]
            ev = etc_v[sl]
            gidx_v[bsl] = c * _RN + ev * _N + sv
            tgtb_v[bsl] = tv
            combb_v[bsl] = tv * _R + ev
        pltpu.async_copy(cnt_sh.at[combb_v], cntb_v, sem).wait()
        for g in range(_B // 16):
            bsl = pl.ds(g * 16, 16)
            normb_v[bsl] = 1.0 / jnp.maximum(cntb_v[bsl], 1.0)
        pltpu.async_copy(xw_hbm.at[gidx_v], rows_v, sem).wait()
        for g in range(_B // 16):
            n16 = normb_v[pl.ds(g * 16, 16)]
            for j in range(16):
                nb = n16[j]
                e = g * 16 + j
                for f in range(8):
                    fs = pl.ds(f * 16, 16)
                    rows_v[e, fs] = rows_v[e, fs] * nb
        pltpu.sync_copy(rows_v, acc_sh.at[tgtb_v], add=True)
        return carry

    lax.fori_loop(0, _NBLK, blk_step, 0)
    plsc.subcore_barrier()
    pltpu.sync_copy(acc_sh.at[nslice], agg_hbm.at[c, nslice])


def _sc1(src, tgt, et, xw_t):
    zrow = jnp.zeros((_NP, 128), jnp.float32)
    z1d = jnp.zeros((_CF,), jnp.float32)
    mesh = plsc.VectorSubcoreMesh(core_axis_name="c", subcore_axis_name="s")
    f = pl.kernel(
        _sc1_body,
        out_type=jax.ShapeDtypeStruct((2, _NP, 128), jnp.float32),
        mesh=mesh,
        compiler_params=pltpu.CompilerParams(needs_layout_passes=False),
        scratch_types=[
            pltpu.VMEM((_B,), jnp.int32),
            pltpu.VMEM((_EC,), jnp.int32),
            pltpu.VMEM((_EC,), jnp.int32),
            pltpu.VMEM((_B,), jnp.int32),
            pltpu.VMEM((_B,), jnp.int32),
            pltpu.VMEM((_B,), jnp.int32),
            pltpu.VMEM((_B,), jnp.float32),
            pltpu.VMEM((_B,), jnp.float32),
            pltpu.VMEM((_B,), jnp.float32),
            pltpu.VMEM((_B, 128), jnp.float32),
            pltpu.VMEM_SHARED((_NP, 128), jnp.float32),
            pltpu.VMEM_SHARED((_CF,), jnp.float32),
            pltpu.SemaphoreType.DMA,
        ],
    )
    return f(src, tgt, et, xw_t, zrow, z1d)


def kernel(x, edge_index, edge_type, W_rgcn, root, b_rgcn, Wq, bq, Wk, bk,
           Wv, bv, Wskip, bskip, Wres, bres, gamma, beta):
    src = edge_index[0]
    tgt = edge_index[1]

    xw_t, hpre = _mm1(x, W_rgcn, root, b_rgcn)

    agg = _sc1(src, tgt, edge_type, xw_t)
    a0 = agg[0, :_N]
    a1 = agg[1, :_N]

    q, k, v, base = _mm2(a0, a1, hpre, Wq, bq, Wk, bk, Wv, bv,
                         Wskip, bskip, Wres, bres)

    qf = jnp.concatenate([q[0], q[1]], axis=1)
    kf = jnp.concatenate([k[0], k[1]], axis=1)
    vf = jnp.concatenate([v[0], v[1]], axis=1)
    score = jnp.sum(qf[tgt] * kf[src], axis=-1) * (1.0 / jnp.sqrt(jnp.float32(_D)))
    ex = jnp.exp(score)
    denom = jax.ops.segment_sum(ex, tgt, num_segments=_N)
    numer = jax.ops.segment_sum(ex[:, None] * vf[src], tgt, num_segments=_N)
    n0, n1 = numer[:, :128], numer[:, 128:]

    o, bsum, bsq = _fin1(n0, n1, denom.reshape(_N, 1), base)
    return _fin2(o, bsum, bsq, gamma, beta)
